# baseline (speedup 1.0000x reference)
import functools

import jax
import jax.numpy as jnp
from jax import lax
from jax.experimental import pallas as pl
from jax.experimental.pallas import tpu as pltpu
from jax.experimental.pallas import tpu_sc as plsc

NC = 2
NS = 16
NW = NC * NS
RB = 8


def _body(idx_hbm, table_hbm, out_hbm, idx_v, rows0, rows1, out0, out1,
          gsem, wsem):
    wid = lax.axis_index("s") * NC + lax.axis_index("c")
    batch, hist = idx_hbm.shape
    rows_per_w = batch // NW
    n_chunks = rows_per_w // RB
    base_row = wid * rows_per_w

    pltpu.sync_copy(idx_hbm.at[pl.ds(base_row, rows_per_w)], idx_v)

    iota = lax.iota(jnp.int32, 16)
    ce0 = iota * 2
    co0 = ce0 + 1
    ce1 = ce0 + 32
    co1 = co0 + 32

    def gather(c, rows, start):
        for h in range(RB):
            cp = pltpu.make_async_copy(
                table_hbm.at[idx_v.at[c * RB + h]],
                rows.at[pl.ds(h * hist, hist)], gsem)
            cp.start() if start else cp.wait()

    def write(c, out, start):
        cp = pltpu.make_async_copy(
            out, out_hbm.at[pl.ds(base_row + c * RB, RB)], wsem)
        cp.start() if start else cp.wait()

    gather(0, rows0, True)
    gather(1, rows1, True)

    @pl.loop(0, n_chunks, step=2)
    def _super(kk):
        for rows, out, b in ((rows0, out0, 0), (rows1, out1, 1)):
            k = kk + b
            gather(k, rows, False)

            @pl.when(k >= 2)
            def _():
                write(k - 2, out, False)

            @pl.loop(0, RB)
            def _brow(i):
                @pl.loop(0, hist, unroll=10)
                def _row(j):
                    rvec = jnp.broadcast_to(i * hist + j, (16,))
                    ea0 = plsc.load_gather(rows, [rvec, ce0])
                    eb0 = plsc.load_gather(rows, [rvec, co0])
                    out[i, j, pl.ds(0, 32)] = plsc.pack(
                        ea0, eb0, format=plsc.PackFormat.INTERLEAVED)
                    ea1 = plsc.load_gather(rows, [rvec, ce1])
                    eb1 = plsc.load_gather(rows, [rvec, co1])
                    out[i, j, pl.ds(32, 32)] = plsc.pack(
                        ea1, eb1, format=plsc.PackFormat.INTERLEAVED)

            write(k, out, True)

            @pl.when(k + 2 < n_chunks)
            def _():
                gather(k + 2, rows, True)

    write(n_chunks - 2, out0, False)
    write(n_chunks - 1, out1, False)


def kernel(x, weight):
    batch, hist = x.shape
    d = weight.shape[1]
    assert batch % (NW * RB) == 0

    run = functools.partial(
        pl.kernel,
        out_type=jax.ShapeDtypeStruct((batch, hist, d), jnp.bfloat16),
        mesh=plsc.VectorSubcoreMesh(core_axis_name="c", subcore_axis_name="s"),
        compiler_params=pltpu.CompilerParams(
            needs_layout_passes=False, use_tc_tiling_on_sc=False),
        scratch_types=[
            pltpu.VMEM((batch // NW, hist), jnp.int32),
            pltpu.VMEM((RB * hist, d), jnp.float32),
            pltpu.VMEM((RB * hist, d), jnp.float32),
            pltpu.VMEM((RB, hist, d), jnp.bfloat16),
            pltpu.VMEM((RB, hist, d), jnp.bfloat16),
            pltpu.SemaphoreType.DMA,
            pltpu.SemaphoreType.DMA,
        ],
    )(_body)
    return run(x, weight)


# speedup vs baseline: 1.1070x; 1.1070x over previous
import functools

import jax
import jax.numpy as jnp
from jax import lax
from jax.experimental import pallas as pl
from jax.experimental.pallas import tpu as pltpu
from jax.experimental.pallas import tpu_sc as plsc

NC = 2
NS = 16
NW = NC * NS
BB = 128


def _body(xt_hbm, wp_hbm, out_hbm, idx_v, runs0, runs1, out0, out1,
          gsem, wsem):
    wid = lax.axis_index("s") * NC + lax.axis_index("c")
    hist, batch = xt_hbm.shape
    b_per_w = batch // NW
    nbb = b_per_w // BB
    n_tasks = hist * nbb
    base_b = wid * b_per_w

    pltpu.sync_copy(xt_hbm.at[pl.ds(0, hist), pl.ds(base_b, b_per_w)], idx_v)

    iota = lax.iota(jnp.int32, 16)
    diag = [(t + iota) & 15 for t in range(16)]

    def gather(c, runs, start):
        j = c // nbb
        r = c - j * nbb
        cp = pltpu.make_async_copy(
            wp_hbm.at[idx_v.at[j, pl.ds(r * BB, BB)]], runs, gsem)
        cp.start() if start else cp.wait()

    def write(c, out, start):
        j = c // nbb
        r = c - j * nbb
        cp = pltpu.make_async_copy(
            out, out_hbm.at[j, pl.ds(0, 32), pl.ds(base_b + r * BB, BB)],
            wsem)
        cp.start() if start else cp.wait()

    def compute(runs, out):
        @pl.loop(0, BB // 16)
        def _grp(g):
            rg = g * 16 + iota
            for t in range(16):
                for s in range(2):
                    col = 2 * diag[t] + 32 * s
                    ea = plsc.load_gather(runs, [rg, col])
                    eb = plsc.load_gather(runs, [rg, col + 1])
                    w = plsc.bitcast(
                        plsc.pack(ea, eb, format=plsc.PackFormat.INTERLEAVED),
                        jnp.int32)
                    plsc.store_scatter(out, [diag[t] + 16 * s, rg], w)

    gather(0, runs0, True)
    gather(1, runs1, True)

    @pl.loop(0, n_tasks, step=2)
    def _super(kk):
        for runs, out, b in ((runs0, out0, 0), (runs1, out1, 1)):
            k = kk + b
            gather(k, runs, False)

            @pl.when(k >= 2)
            def _():
                write(k - 2, out, False)

            compute(runs, out)
            write(k, out, True)

            @pl.when(k + 2 < n_tasks)
            def _():
                gather(k + 2, runs, True)

    write(n_tasks - 2, out0, False)
    write(n_tasks - 1, out1, False)


def kernel(x, weight):
    batch, hist = x.shape
    v, d = weight.shape
    assert batch % (NW * BB) == 0 and d == 64

    xt = x.T
    wp = jnp.pad(weight, ((0, 0), (0, 128 - d)))

    run = functools.partial(
        pl.kernel,
        out_type=jax.ShapeDtypeStruct((hist, d // 2, batch), jnp.int32),
        mesh=plsc.VectorSubcoreMesh(core_axis_name="c", subcore_axis_name="s"),
        compiler_params=pltpu.CompilerParams(
            needs_layout_passes=False, use_tc_tiling_on_sc=True),
        scratch_types=[
            pltpu.VMEM((hist, batch // NW), jnp.int32),
            pltpu.VMEM((BB, 128), jnp.float32),
            pltpu.VMEM((BB, 128), jnp.float32),
            pltpu.VMEM((d // 2, BB), jnp.int32),
            pltpu.VMEM((d // 2, BB), jnp.int32),
            pltpu.SemaphoreType.DMA,
            pltpu.SemaphoreType.DMA,
        ],
    )(_body)
    y = run(xt, wp)
    yb = jax.lax.bitcast_convert_type(y, jnp.bfloat16)
    return yb.transpose(2, 0, 1, 3).reshape(batch, hist, d)


# speedup vs baseline: 1.6375x; 1.4792x over previous
import functools

import jax
import jax.numpy as jnp
from jax import lax
from jax.experimental import pallas as pl
from jax.experimental.pallas import tpu as pltpu
from jax.experimental.pallas import tpu_sc as plsc

NC = 2
NS = 16
NW = NC * NS
BB = 128


def _body(xt_hbm, wq_hbm, out_hbm, idx_v, pb0, pb1, runs0, runs1, out0, out1,
          gsem, wsem):
    wid = lax.axis_index("s") * NC + lax.axis_index("c")
    hist, batch = xt_hbm.shape
    b_per_w = batch // NW
    nbb = b_per_w // BB
    n_tasks = hist * nbb
    base_b = wid * b_per_w

    pltpu.sync_copy(xt_hbm.at[pl.ds(0, hist), pl.ds(base_b, b_per_w)], idx_v)

    iota = lax.iota(jnp.int32, 16)
    diag = [(t + iota) & 15 for t in range(16)]

    def task(c):
        j = c // nbb
        return j, c - j * nbb

    def gather(c, pb, runs, start):
        j, r = task(c)
        if start:
            for h in range(BB // 16):
                v = idx_v[j, pl.ds(r * BB + 16 * h, 16)]
                pb[pl.ds(16 * h, 16)] = lax.shift_right_logical(v, 1)
        cp = pltpu.make_async_copy(wq_hbm.at[pb], runs, gsem)
        cp.start() if start else cp.wait()

    def write(c, out, start):
        j, r = task(c)
        cp = pltpu.make_async_copy(
            out,
            out_hbm.bitcast(jnp.int32).at[
                j, pl.ds(0, 32), pl.ds(base_b + r * BB, BB)],
            wsem)
        cp.start() if start else cp.wait()

    def compute(c, runs, out):
        j, r = task(c)

        @pl.loop(0, BB // 16)
        def _grp(g):
            rg = g * 16 + iota
            iv = idx_v[j, pl.ds(r * BB + 16 * g, 16)]
            pv64 = (iv & 1) << 6
            for t in range(16):
                for s in range(2):
                    cb = pv64 + (2 * diag[t] + 32 * s)
                    ea = plsc.load_gather(runs, [rg, cb])
                    eb = plsc.load_gather(runs, [rg, cb + 1])
                    w = plsc.bitcast(
                        plsc.pack(ea, eb, format=plsc.PackFormat.INTERLEAVED),
                        jnp.int32)
                    plsc.store_scatter(out, [diag[t] + 16 * s, rg], w)

    gather(0, pb0, runs0, True)
    gather(1, pb1, runs1, True)

    @pl.loop(0, n_tasks, step=2)
    def _super(kk):
        for pb, runs, out, b in ((pb0, runs0, out0, 0), (pb1, runs1, out1, 1)):
            k = kk + b
            gather(k, pb, runs, False)

            @pl.when(k >= 2)
            def _():
                write(k - 2, out, False)

            compute(k, runs, out)
            write(k, out, True)

            @pl.when(k + 2 < n_tasks)
            def _():
                gather(k + 2, pb, runs, True)

    write(n_tasks - 2, out0, False)
    write(n_tasks - 1, out1, False)


def kernel(x, weight):
    batch, hist = x.shape
    v, d = weight.shape
    assert batch % (NW * BB) == 0 and d == 64

    xt = x.T
    wq = weight.reshape(v * d // 128, 128)

    run = functools.partial(
        pl.kernel,
        out_type=jax.ShapeDtypeStruct((hist, d, batch), jnp.bfloat16),
        mesh=plsc.VectorSubcoreMesh(core_axis_name="c", subcore_axis_name="s"),
        compiler_params=pltpu.CompilerParams(
            needs_layout_passes=False, use_tc_tiling_on_sc=True),
        scratch_types=[
            pltpu.VMEM((hist, batch // NW), jnp.int32),
            pltpu.VMEM((BB,), jnp.int32),
            pltpu.VMEM((BB,), jnp.int32),
            pltpu.VMEM((BB, 128), jnp.float32),
            pltpu.VMEM((BB, 128), jnp.float32),
            pltpu.VMEM((d // 2, BB), jnp.int32),
            pltpu.VMEM((d // 2, BB), jnp.int32),
            pltpu.SemaphoreType.DMA,
            pltpu.SemaphoreType.DMA,
        ],
    )(_body)
    y = run(xt, wq)
    return y.transpose(2, 0, 1)
